# baseline (speedup 1.0000x reference)
import functools
import jax
import jax.numpy as jnp
from jax import lax
from jax.experimental import pallas as pl
from jax.experimental.pallas import tpu as pltpu
from jax.experimental.pallas import tpu_sc as plsc

_V = 100000
_B = 1024
_FILL = 0.1 / (_V - 2)
_NC = 1
_NS = 16
_NW = _NC * _NS
_RPW = _B // _NW
_L = 16


def _sc_body(trg_hbm, out_hbm, trgv, tmpl, patterns, sem_rows, sem_pat):
    wid = lax.axis_index("s") * _NC + lax.axis_index("c")
    base = wid * _RPW
    pltpu.sync_copy(trg_hbm.at[pl.ds(base, _RPW)], trgv)

    lane = lax.iota(jnp.int32, _L)
    fillv = jnp.full((_L,), _FILL, jnp.float32)

    def fill_body(i, _):
        tmpl[pl.ds(i * _L, _L)] = fillv
        return 0

    lax.fori_loop(1, _V // _L, fill_body, 0)
    tmpl[pl.ds(0, _L)] = jnp.where(lane == 0, 0.0, fillv)

    chunks = [trgv[pl.ds(c * _L, _L)] for c in range(_RPW // _L)]
    ts = [chunks[r // _L][r % _L] for r in range(_RPW)]

    for r in range(_RPW):
        @pl.when(ts[r] != 0)
        def _(r=r):
            pltpu.async_copy(tmpl, out_hbm.at[base + r], sem_rows)
    for r in range(_RPW):
        @pl.when(ts[r] != 0)
        def _(r=r):
            pltpu.make_async_copy(tmpl, out_hbm.at[base + r], sem_rows).wait()

    npad = ts[0] * 0
    for r in range(_RPW):
        npad = npad + (ts[r] == 0).astype(jnp.int32)

    @pl.when(npad > 0)
    def _():
        zv = jnp.zeros((_L,), jnp.float32)

        def zfill(i, _):
            tmpl[pl.ds(i * _L, _L)] = zv
            return 0

        lax.fori_loop(0, _V // _L, zfill, 0)
        for r in range(_RPW):
            @pl.when(ts[r] == 0)
            def _(r=r):
                pltpu.async_copy(tmpl, out_hbm.at[base + r], sem_rows)
        for r in range(_RPW):
            @pl.when(ts[r] == 0)
            def _(r=r):
                pltpu.make_async_copy(tmpl, out_hbm.at[base + r], sem_rows).wait()

    def _rep_lane(chunk, k):
        m = jnp.where(lane == k, chunk, 0)
        pre = jnp.cumsum(m)
        suf = jnp.flip(jnp.cumsum(jnp.flip(m)))
        return pre + suf - m

    for r in range(_RPW):
        t_rep = _rep_lane(chunks[r // _L], r % _L)
        loc = t_rep - (t_rep // _L) * _L
        pat = jnp.where(lane == loc, 0.9, fillv)
        pat = jnp.where((t_rep < _L) & (lane == 0), 0.0, pat)
        patterns[pl.ds(r * _L, _L)] = pat

    for r in range(_RPW):
        @pl.when(ts[r] != 0)
        def _(r=r):
            wstart = pl.multiple_of((ts[r] // _L) * _L, 8)
            pltpu.async_copy(
                patterns.at[pl.ds(r * _L, _L)],
                out_hbm.at[base + r].at[pl.ds(wstart, _L)],
                sem_pat,
            )
    for r in range(_RPW):
        @pl.when(ts[r] != 0)
        def _(r=r):
            wstart = pl.multiple_of((ts[r] // _L) * _L, 8)
            pltpu.make_async_copy(
                patterns.at[pl.ds(r * _L, _L)],
                out_hbm.at[base + r].at[pl.ds(wstart, _L)],
                sem_pat,
            ).wait()


_mesh = plsc.VectorSubcoreMesh(core_axis_name="c", subcore_axis_name="s", num_cores=1)

_sc_call = functools.partial(
    pl.kernel,
    mesh=_mesh,
    compiler_params=pltpu.CompilerParams(needs_layout_passes=False, use_tc_tiling_on_sc=True),
    out_type=jax.ShapeDtypeStruct((_B, _V), jnp.float32),
    scratch_types=[
        pltpu.VMEM((_RPW,), jnp.int32),
        pltpu.VMEM((_V,), jnp.float32),
        pltpu.VMEM((_RPW * _L,), jnp.float32),
        pltpu.SemaphoreType.DMA,
        pltpu.SemaphoreType.DMA,
    ],
)(_sc_body)


def kernel(trg_token_ids_batch):
    trg = trg_token_ids_batch.reshape(_B)
    return _sc_call(trg)


# speedup vs baseline: 3.6937x; 3.6937x over previous
import functools
import jax
import jax.numpy as jnp
from jax import lax
from jax.experimental import pallas as pl
from jax.experimental.pallas import tpu as pltpu
from jax.experimental.pallas import tpu_sc as plsc

_V = 100000
_B = 1024
_SMOOTH = 0.1
_CONF = 1.0 - _SMOOTH
_FILL = _SMOOTH / (_V - 2)

_NC = 2
_NS = 16
_L = 16

_STRIPS = _V // 8
_BASE_STRIPS = _STRIPS // (_NC * _NS)
_EXTRA = _STRIPS - _BASE_STRIPS * _NC * _NS
_TC = 12
_TR = _TC * 8
_NFULL = _BASE_STRIPS // _TC
_HALF0 = (_BASE_STRIPS + 1) * _NS * 8


def _sc_body(trg_hbm, out_hbm, trgv, tmpl, prow, zrow, patbuf, sem_s, sem_p):
    c = lax.axis_index("c")
    s = lax.axis_index("s")
    wid = c * _NS + s
    lane = lax.iota(jnp.int32, _L)
    fillv = jnp.full((_L,), _FILL, jnp.float32)
    zerov = jnp.zeros((_L,), jnp.float32)
    confv = jnp.full((_L,), _CONF, jnp.float32)

    pltpu.sync_copy(trg_hbm, trgv)

    for cb in range(_B // _L):
        t = trgv[pl.ds(cb * _L, _L)]
        prow[pl.ds(cb * _L, _L)] = jnp.where(t == 0, 0.0, fillv)
        zrow[pl.ds(cb * _L, _L)] = zerov

    def trow(v, _):
        def tcol(cb, _):
            tmpl[v, pl.ds(cb * _L, _L)] = prow[pl.ds(cb * _L, _L)]
            return 0

        lax.fori_loop(0, _B // _L, tcol, 0)
        return 0

    lax.fori_loop(0, _TR, trow, 0)

    cnt = _BASE_STRIPS + (wid < _EXTRA).astype(jnp.int32)
    start = wid * _BASE_STRIPS + jnp.minimum(wid, _EXTRA)

    def _chunk_ref(k):
        v0 = pl.multiple_of((start + k * _TC) * 8, 8)
        return out_hbm.at[pl.ds(v0, _TR), :]

    for k in range(_NFULL):
        pltpu.async_copy(tmpl, _chunk_ref(k), sem_s)

    tail0 = pl.multiple_of((start + _NFULL * _TC) * 8, 8)
    tail_long = cnt == _BASE_STRIPS + 1

    @pl.when(tail_long)
    def _():
        pltpu.async_copy(
            tmpl.at[pl.ds(0, 56), :], out_hbm.at[pl.ds(tail0, 56), :], sem_s)

    @pl.when(jnp.logical_not(tail_long))
    def _():
        pltpu.async_copy(
            tmpl.at[pl.ds(0, 48), :], out_hbm.at[pl.ds(tail0, 48), :], sem_s)

    for k in range(_NFULL):
        pltpu.make_async_copy(tmpl, _chunk_ref(k), sem_s).wait()

    @pl.when(tail_long)
    def _():
        pltpu.make_async_copy(
            tmpl.at[pl.ds(0, 56), :], out_hbm.at[pl.ds(tail0, 56), :], sem_s
        ).wait()

    @pl.when(jnp.logical_not(tail_long))
    def _():
        pltpu.make_async_copy(
            tmpl.at[pl.ds(0, 48), :], out_hbm.at[pl.ds(tail0, 48), :], sem_s
        ).wait()

    @pl.when(wid == 0)
    def _():
        pltpu.async_copy(zrow, out_hbm.at[0], sem_p)
        pltpu.make_async_copy(zrow, out_hbm.at[0], sem_p).wait()

    plsc.subcore_barrier()

    def _rep_lane(chunk, k):
        m = jnp.where(lane == k, chunk, 0)
        pre = jnp.cumsum(m)
        suf = jnp.flip(jnp.cumsum(jnp.flip(m)))
        return pre + suf - m

    lo = c * _HALF0
    hi = lo + jnp.where(c == 0, _HALF0, _V - _HALF0)
    npat = _B // _NS

    for j in range(npat):
        b = s * npat + j
        cb = trgv[pl.ds(s * npat + (j // _L) * _L, _L)]
        tb = cb[j % _L]

        @pl.when((tb >= lo) & (tb < hi) & (tb != 0))
        def _(j=j, cb=cb, tb=tb, b=b):
            trep = _rep_lane(cb, j % _L)
            pat = jnp.where(cb == trep, confv, fillv)
            pat = jnp.where(cb == 0, 0.0, pat)
            patbuf[pl.ds(j * _L, _L)] = pat
            b16 = pl.multiple_of(s * npat + (j // _L) * _L, 8)
            pltpu.async_copy(
                patbuf.at[pl.ds(j * _L, _L)],
                out_hbm.at[tb, pl.ds(b16, _L)],
                sem_p,
            )

    for j in range(npat):
        cb = trgv[pl.ds(s * npat + (j // _L) * _L, _L)]
        tb = cb[j % _L]

        @pl.when((tb >= lo) & (tb < hi) & (tb != 0))
        def _(j=j, tb=tb):
            b16 = pl.multiple_of(s * npat + (j // _L) * _L, 8)
            pltpu.make_async_copy(
                patbuf.at[pl.ds(j * _L, _L)],
                out_hbm.at[tb, pl.ds(b16, _L)],
                sem_p,
            ).wait()


_mesh = plsc.VectorSubcoreMesh(core_axis_name="c", subcore_axis_name="s")

_sc_call = functools.partial(
    pl.kernel,
    mesh=_mesh,
    compiler_params=pltpu.CompilerParams(
        needs_layout_passes=False, use_tc_tiling_on_sc=True),
    out_type=jax.ShapeDtypeStruct((_V, _B), jnp.float32),
    scratch_types=[
        pltpu.VMEM((_B,), jnp.int32),
        pltpu.VMEM((_TR, _B), jnp.float32),
        pltpu.VMEM((_B,), jnp.float32),
        pltpu.VMEM((_B,), jnp.float32),
        pltpu.VMEM((_B // _NS * _L,), jnp.float32),
        pltpu.SemaphoreType.DMA,
        pltpu.SemaphoreType.DMA,
    ],
)(_sc_body)


def kernel(trg_token_ids_batch):
    trg = trg_token_ids_batch.reshape(_B)
    return jnp.transpose(_sc_call(trg))


# speedup vs baseline: 4.1554x; 1.1250x over previous
import functools
import jax
import jax.numpy as jnp
from jax import lax
from jax.experimental import pallas as pl
from jax.experimental.pallas import tpu as pltpu
from jax.experimental.pallas import tpu_sc as plsc

_V = 100000
_B = 1024
_SMOOTH = 0.1
_CONF = 1.0 - _SMOOTH
_FILL = _SMOOTH / (_V - 2)

_NC = 2
_NS = 16
_L = 16

_STRIPS = _V // 8
_BASE_STRIPS = _STRIPS // (_NC * _NS)
_EXTRA = _STRIPS - _BASE_STRIPS * _NC * _NS
_HALF0 = (_BASE_STRIPS + 1) * _NS * 8


def _sc_body(trg_hbm, out_hbm, trgv, tmpl, prow, zrow, patbuf, sem_s, sem_p):
    c = lax.axis_index("c")
    s = lax.axis_index("s")
    wid = c * _NS + s
    lane = lax.iota(jnp.int32, _L)
    fillv = jnp.full((_L,), _FILL, jnp.float32)
    zerov = jnp.zeros((_L,), jnp.float32)
    confv = jnp.full((_L,), _CONF, jnp.float32)

    pltpu.sync_copy(trg_hbm, trgv)

    for cb in range(_B // _L):
        t = trgv[pl.ds(cb * _L, _L)]
        pv = jnp.where(t == 0, 0.0, fillv)
        prow[pl.ds(cb * _L, _L)] = pv
        zrow[pl.ds(cb * _L, _L)] = zerov
        for v in range(8):
            tmpl[v, pl.ds(cb * _L, _L)] = pv

    cnt = _BASE_STRIPS + (wid < _EXTRA).astype(jnp.int32)
    start = wid * _BASE_STRIPS + jnp.minimum(wid, _EXTRA)

    def _issue(k, _):
        v0 = pl.multiple_of((start + k) * 8, 8)
        pltpu.async_copy(tmpl, out_hbm.at[pl.ds(v0, 8), :], sem_s)
        return 0

    def _drain(k, _):
        v0 = pl.multiple_of((start + k) * 8, 8)
        pltpu.make_async_copy(tmpl, out_hbm.at[pl.ds(v0, 8), :], sem_s).wait()
        return 0

    lax.fori_loop(0, cnt, _issue, 0)
    lax.fori_loop(0, cnt, _drain, 0)

    @pl.when(wid == 0)
    def _():
        pltpu.async_copy(zrow, out_hbm.at[0], sem_p)
        pltpu.make_async_copy(zrow, out_hbm.at[0], sem_p).wait()

    plsc.subcore_barrier()

    def _rep_lane(chunk, k):
        m = jnp.where(lane == k, chunk, 0)
        pre = jnp.cumsum(m)
        suf = jnp.flip(jnp.cumsum(jnp.flip(m)))
        return pre + suf - m

    lo = c * _HALF0
    hi = lo + jnp.where(c == 0, _HALF0, _V - _HALF0)
    npat = _B // _NS

    for j in range(npat):
        cb = trgv[pl.ds(s * npat + (j // _L) * _L, _L)]
        tb = cb[j % _L]

        @pl.when((tb >= lo) & (tb < hi) & (tb != 0))
        def _(j=j, cb=cb, tb=tb):
            trep = _rep_lane(cb, j % _L)
            pat = jnp.where(cb == trep, confv, fillv)
            pat = jnp.where(cb == 0, 0.0, pat)
            patbuf[pl.ds(j * _L, _L)] = pat
            b16 = pl.multiple_of(s * npat + (j // _L) * _L, 8)
            pltpu.async_copy(
                patbuf.at[pl.ds(j * _L, _L)],
                out_hbm.at[tb, pl.ds(b16, _L)],
                sem_p,
            )

    for j in range(npat):
        cb = trgv[pl.ds(s * npat + (j // _L) * _L, _L)]
        tb = cb[j % _L]

        @pl.when((tb >= lo) & (tb < hi) & (tb != 0))
        def _(j=j, tb=tb):
            b16 = pl.multiple_of(s * npat + (j // _L) * _L, 8)
            pltpu.make_async_copy(
                patbuf.at[pl.ds(j * _L, _L)],
                out_hbm.at[tb, pl.ds(b16, _L)],
                sem_p,
            ).wait()


_mesh = plsc.VectorSubcoreMesh(core_axis_name="c", subcore_axis_name="s")

_sc_call = functools.partial(
    pl.kernel,
    mesh=_mesh,
    compiler_params=pltpu.CompilerParams(
        needs_layout_passes=False, use_tc_tiling_on_sc=True),
    out_type=jax.ShapeDtypeStruct((_V, _B), jnp.float32),
    scratch_types=[
        pltpu.VMEM((_B,), jnp.int32),
        pltpu.VMEM((8, _B), jnp.float32),
        pltpu.VMEM((_B,), jnp.float32),
        pltpu.VMEM((_B,), jnp.float32),
        pltpu.VMEM((_B // _NS * _L,), jnp.float32),
        pltpu.SemaphoreType.DMA,
        pltpu.SemaphoreType.DMA,
    ],
)(_sc_body)


def kernel(trg_token_ids_batch):
    trg = trg_token_ids_batch.reshape(_B)
    return jnp.transpose(_sc_call(trg))
